# baseline (speedup 1.0000x reference)
import dataclasses
import functools

import jax
import jax.numpy as jnp
from jax import lax
from jax.experimental import pallas as pl
from jax.experimental.pallas import tpu as pltpu
from jax.experimental.pallas import tpu_sc as plsc

N = 100000
C = 512
HID = 128
BS = 8
L = 16
NC = 2
NS = 16
NW = NC * NS
CSL = C // L
HSL = HID // L
HCSL = CSL // 2

BLK = 80
NBLK = N // BLK
BSL = BLK // L

TBLK = 640
TGRID = 64
TCA = 58880
TCB = TCA + TGRID * TBLK
NBLK_SC = TCA // BLK + (N - TCB) // BLK

_I0 = functools.partial(jnp.full, (L,))


def _mesh():
    return plsc.VectorSubcoreMesh(core_axis_name="core", subcore_axis_name="subcore")


def _cparams():
    cp = pltpu.CompilerParams()
    if "needs_layout_passes" in pltpu.CompilerParams.__dataclass_fields__:
        cp = dataclasses.replace(cp, needs_layout_passes=False)
    return cp


def _fr(ref):
    return ref.at[tuple(pl.ds(0, s) for s in ref.shape)]


def _wid():
    return lax.axis_index("core") * NS + lax.axis_index("subcore")


def _nblocks_for(wid, nblk=NBLK):
    return (nblk - 1 - wid) // NW + 1


def _block_minmax(bb):
    vs = [bb[pl.ds(k * L, L)] for k in range(BSL)]
    bx, bn = vs[0], vs[0]
    for v in vs[1:]:
        bx = jnp.maximum(bx, v)
        bn = jnp.minimum(bn, v)
    return jnp.min(bn), jnp.max(bx)


def _pool(feats, bidx1):
    @functools.partial(
        pl.kernel,
        out_type=(jax.ShapeDtypeStruct((NW * BS * C,), jnp.float32),
                  jax.ShapeDtypeStruct((NW * L,), jnp.float32)),
        mesh=_mesh(),
        compiler_params=_cparams(),
        scratch_types=[pltpu.VMEM((BS * C,), jnp.float32),
                       pltpu.VMEM((L,), jnp.float32),
                       pltpu.VMEM((BLK, C), jnp.float32),
                       pltpu.VMEM((BLK, C), jnp.float32),
                       pltpu.VMEM((BLK,), jnp.int32),
                       pltpu.VMEM((BLK,), jnp.int32),
                       pltpu.SemaphoreType.DMA,
                       pltpu.SemaphoreType.DMA],
    )
    def k(feats_hbm, bidx_hbm, part_hbm, cnt_hbm,
          acc, cnt, fb0, fb1, bb0, bb1, sem0, sem1):
        zero = jnp.zeros((L,), jnp.float32)

        @pl.loop(0, BS * C, step=L)
        def _(i):
            acc[pl.ds(i, L)] = zero

        cnt[...] = zero
        iota = lax.iota(jnp.int32, L)
        one = jnp.float32(1.0)
        zf = jnp.float32(0.0)
        wid = _wid()
        nb = _nblocks_for(wid, NBLK_SC)
        head = TCA // BLK

        def start_in(j, fb, bb, sem):
            bid = wid + j * NW
            base = pl.multiple_of(
                jnp.where(bid < head, bid * BLK, TCB + (bid - head) * BLK), 8)
            pltpu.async_copy(feats_hbm.at[pl.ds(base, BLK)], fb, sem)
            pltpu.async_copy(bidx_hbm.at[pl.ds(base, BLK)], bb, sem)

        def wait_in(fb, bb, sem):
            pltpu.make_async_copy(feats_hbm.at[pl.ds(0, BLK)], fb, sem).wait()
            pltpu.make_async_copy(bidx_hbm.at[pl.ds(0, BLK)], bb, sem).wait()

        def compute(fb, bb):
            mn, mx = _block_minmax(bb)

            def fast():
                off = mx * C
                for half in range(2):
                    hof = half * HCSL * L

                    def rbody(r, sums):
                        return tuple(
                            sums[t] + fb[r, pl.ds(hof + t * L, L)]
                            for t in range(HCSL))

                    sums = lax.fori_loop(
                        0, BLK, rbody, (zero,) * HCSL)
                    for t in range(HCSL):
                        plsc.addupdate(acc.at[pl.ds(off + hof + t * L, L)],
                                       sums[t])
                plsc.addupdate(cnt.at[pl.ds(0, L)],
                               jnp.where(iota == mx, jnp.float32(BLK), zf))

            def slow():
                @pl.loop(0, BLK)
                def _(r):
                    b16 = plsc.load_gather(_fr(bb), [_I0(r, jnp.int32)])
                    addr = b16 * C + iota
                    for j in range(CSL):
                        plsc.addupdate_scatter(_fr(acc), [addr + (j * L)],
                                               fb[r, pl.ds(j * L, L)])
                    plsc.addupdate(cnt.at[pl.ds(0, L)],
                                   jnp.where(iota == b16, one, zf))

            lax.cond(mn == mx, fast, slow)

        start_in(0, fb0, bb0, sem0)
        start_in(1, fb1, bb1, sem1)

        @pl.loop(0, nb // 2)
        def _(p):
            j0 = 2 * p
            wait_in(fb0, bb0, sem0)
            compute(fb0, bb0)

            @pl.when(j0 + 2 < nb)
            def _():
                start_in(j0 + 2, fb0, bb0, sem0)

            wait_in(fb1, bb1, sem1)
            compute(fb1, bb1)

            @pl.when(j0 + 3 < nb)
            def _():
                start_in(j0 + 3, fb1, bb1, sem1)

        @pl.when(nb % 2 == 1)
        def _():
            wait_in(fb0, bb0, sem0)
            compute(fb0, bb0)

        pbase = pl.multiple_of(wid * (BS * C), 8)
        pltpu.sync_copy(acc, part_hbm.at[pl.ds(pbase, BS * C)])
        cbase = pl.multiple_of(wid * L, 8)
        pltpu.sync_copy(cnt, cnt_hbm.at[pl.ds(cbase, L)])

    return k(feats, bidx1)


def _pool_tc(feats, bidx2):
    def body(b_ref, f_ref, part_ref, cnt_ref, acc, cacc):
        i = pl.program_id(0)

        @pl.when(i == 0)
        def _():
            acc[...] = jnp.zeros_like(acc)
            cacc[...] = jnp.zeros_like(cacc)

        fx = f_ref[...]
        for b in range(BS):
            mask = (b_ref[...] == b).astype(jnp.float32)
            row = lax.dot_general(mask, fx, (((1,), (0,)), ((), ())),
                                  preferred_element_type=jnp.float32)
            acc[pl.ds(b, 1), :] += row
            cacc[pl.ds(b, 1), :] += jnp.sum(mask)

        @pl.when(i == TGRID - 1)
        def _():
            part_ref[...] = acc[...]
            cnt_ref[...] = cacc[...]

    return pl.pallas_call(
        body,
        grid=(TGRID,),
        in_specs=[pl.BlockSpec((1, TBLK), lambda i: (0, TCA // TBLK + i)),
                  pl.BlockSpec((TBLK, C), lambda i: (TCA // TBLK + i, 0))],
        out_specs=[pl.BlockSpec((BS, C), lambda i: (0, 0)),
                   pl.BlockSpec((BS, HID), lambda i: (0, 0))],
        out_shape=[jax.ShapeDtypeStruct((BS, C), jnp.float32),
                   jax.ShapeDtypeStruct((BS, HID), jnp.float32)],
        scratch_shapes=[pltpu.VMEM((BS, C), jnp.float32),
                        pltpu.VMEM((BS, HID), jnp.float32)],
    )(bidx2, feats)


def _mlp_tc(part, cnt, tc_part, tc_cnt, fc1_w, fc1_b, fc2_w, fc2_b):
    def body(part_ref, cnt_ref, tcp_ref, tcc_ref,
             w1_ref, b1_ref, w2_ref, b2_ref, se_ref):
        pooled = jnp.sum(
            part_ref[...].reshape(NW, BS, C), axis=0) + tcp_ref[...]
        counts = (jnp.sum(cnt_ref[...], axis=0)[:BS]
                  + jnp.sum(tcc_ref[...], axis=1) * (1.0 / HID))
        avg = pooled * (1.0 / jnp.maximum(counts, 1.0))[:, None]
        h = lax.dot_general(avg, w1_ref[...],
                            (((1,), (1,)), ((), ())),
                            preferred_element_type=jnp.float32)
        h = jnp.maximum(h + b1_ref[...][None, :], 0.0)
        s = lax.dot_general(h, w2_ref[...],
                            (((1,), (1,)), ((), ())),
                            preferred_element_type=jnp.float32)
        s = s + b2_ref[...][None, :]
        se_ref[...] = 1.0 / (1.0 + jnp.exp(-s))

    return pl.pallas_call(
        body,
        out_shape=jax.ShapeDtypeStruct((BS, C), jnp.float32),
    )(part.reshape(NW, BS * C), cnt.reshape(NW, L), tc_part, tc_cnt,
      fc1_w, fc1_b, fc2_w, fc2_b)


def _scale(feats, bidx1, se_flat):
    @functools.partial(
        pl.kernel,
        out_type=jax.ShapeDtypeStruct((N, C), jnp.float32),
        mesh=_mesh(),
        compiler_params=_cparams(),
        scratch_types=[pltpu.VMEM((C * BS,), jnp.float32),
                       pltpu.VMEM((BLK, C), jnp.float32),
                       pltpu.VMEM((BLK, C), jnp.float32),
                       pltpu.VMEM((BLK, C), jnp.float32),
                       pltpu.VMEM((BLK,), jnp.int32),
                       pltpu.VMEM((BLK,), jnp.int32),
                       pltpu.VMEM((BLK,), jnp.int32),
                       pltpu.SemaphoreType.DMA,
                       pltpu.SemaphoreType.DMA,
                       pltpu.SemaphoreType.DMA,
                       pltpu.SemaphoreType.DMA,
                       pltpu.SemaphoreType.DMA,
                       pltpu.SemaphoreType.DMA],
    )
    def k(feats_hbm, bidx_hbm, se_hbm, out_hbm,
          sev, fb0, fb1, fb2, bb0, bb1, bb2, si0, si1, si2, so0, so1, so2):
        iota = lax.iota(jnp.int32, L)
        wid = _wid()
        nb = _nblocks_for(wid)

        pltpu.sync_copy(se_hbm, sev)

        bufs = ((fb0, bb0, si0, so0),
                (fb1, bb1, si1, so1),
                (fb2, bb2, si2, so2))

        def start_in(j, buf):
            fb, bb, si, _so = buf
            base = pl.multiple_of((wid + j * NW) * BLK, 8)
            pltpu.async_copy(feats_hbm.at[pl.ds(base, BLK)], fb, si)
            pltpu.async_copy(bidx_hbm.at[pl.ds(base, BLK)], bb, si)

        def wait_in(buf):
            fb, bb, si, _so = buf
            pltpu.make_async_copy(feats_hbm.at[pl.ds(0, BLK)], fb, si).wait()
            pltpu.make_async_copy(bidx_hbm.at[pl.ds(0, BLK)], bb, si).wait()

        def start_out(j, buf):
            fb, _bb, _si, so = buf
            base = pl.multiple_of((wid + j * NW) * BLK, 8)
            pltpu.async_copy(fb, out_hbm.at[pl.ds(base, BLK)], so)

        def wait_out(buf):
            fb, _bb, _si, so = buf
            pltpu.make_async_copy(feats_hbm.at[pl.ds(0, BLK)], fb, so).wait()

        def compute(buf):
            fb, bb = buf[0], buf[1]
            mn, mx = _block_minmax(bb)

            def fast():
                off = mx * C
                for half in range(2):
                    hof = half * HCSL * L
                    ses_regs = tuple(sev[pl.ds(off + hof + t * L, L)]
                                     for t in range(HCSL))

                    @pl.loop(0, BLK)
                    def _(r):
                        for t in range(HCSL):
                            sl = pl.ds(hof + t * L, L)
                            fb[r, sl] = fb[r, sl] * ses_regs[t]

            def slow():
                @pl.loop(0, BLK)
                def _(r):
                    b16 = plsc.load_gather(_fr(bb), [_I0(r, jnp.int32)])
                    addr = b16 * C + iota
                    for j in range(CSL):
                        sl = pl.ds(j * L, L)
                        s16 = plsc.load_gather(_fr(sev), [addr + (j * L)])
                        fb[r, sl] = fb[r, sl] * s16

            lax.cond(mn == mx, fast, slow)

        def step(j, k_static, t):
            cur = bufs[k_static]
            nxt = bufs[(k_static + 2) % 3]
            wait_in(cur)
            compute(cur)
            start_out(j, cur)
            if k_static == 0:
                @pl.when(t > 0)
                def _():
                    wait_out(nxt)
            else:
                wait_out(nxt)

            @pl.when(j + 2 < nb)
            def _():
                start_in(j + 2, nxt)

        start_in(0, bufs[0])
        start_in(1, bufs[1])

        @pl.loop(0, nb // 3)
        def _(t):
            j0 = 3 * t
            step(j0, 0, t)
            step(j0 + 1, 1, t)
            step(j0 + 2, 2, t)

        @pl.when(nb % 3 == 1)
        def _():
            j = nb - 1
            cur = bufs[0]
            wait_in(cur)
            compute(cur)
            start_out(j, cur)
            wait_out(bufs[2])
            wait_out(bufs[0])

        @pl.when(nb % 3 == 0)
        def _():
            wait_out(bufs[2])

    return k(feats, bidx1, se_flat)


def kernel(feats, fc1_w, fc1_b, fc2_w, fc2_b, bidx, batch_size):
    del batch_size
    bidx1 = bidx.astype(jnp.int32).reshape(N)
    part, cnt = _pool(feats, bidx1)
    tc_part, tc_cnt = _pool_tc(feats, bidx1.reshape(1, N))
    se = _mlp_tc(part, cnt, tc_part, tc_cnt, fc1_w, fc1_b, fc2_w, fc2_b)
    return _scale(feats, bidx1, se.reshape(BS * C))


# speedup vs baseline: 1.0250x; 1.0250x over previous
import dataclasses
import functools

import jax
import jax.numpy as jnp
from jax import lax
from jax.experimental import pallas as pl
from jax.experimental.pallas import tpu as pltpu
from jax.experimental.pallas import tpu_sc as plsc

N = 100000
C = 512
HID = 128
BS = 8
L = 16
NC = 2
NS = 16
NW = NC * NS
CSL = C // L
HSL = HID // L
HCSL = CSL // 2

BLK = 80
NBLK = N // BLK
BSL = BLK // L

TBLK = 640
TGRID = 48
TCA = 69120
TCB = TCA + TGRID * TBLK
NBLK_SC = TCA // BLK + (N - TCB) // BLK

_I0 = functools.partial(jnp.full, (L,))


def _mesh():
    return plsc.VectorSubcoreMesh(core_axis_name="core", subcore_axis_name="subcore")


def _cparams():
    cp = pltpu.CompilerParams()
    if "needs_layout_passes" in pltpu.CompilerParams.__dataclass_fields__:
        cp = dataclasses.replace(cp, needs_layout_passes=False)
    return cp


def _fr(ref):
    return ref.at[tuple(pl.ds(0, s) for s in ref.shape)]


def _wid():
    return lax.axis_index("core") * NS + lax.axis_index("subcore")


def _nblocks_for(wid, nblk=NBLK):
    return (nblk - 1 - wid) // NW + 1


def _block_minmax(bb):
    vs = [bb[pl.ds(k * L, L)] for k in range(BSL)]
    bx, bn = vs[0], vs[0]
    for v in vs[1:]:
        bx = jnp.maximum(bx, v)
        bn = jnp.minimum(bn, v)
    return jnp.min(bn), jnp.max(bx)


def _pool(feats, bidx1):
    @functools.partial(
        pl.kernel,
        out_type=(jax.ShapeDtypeStruct((NW * BS * C,), jnp.float32),
                  jax.ShapeDtypeStruct((NW * L,), jnp.float32)),
        mesh=_mesh(),
        compiler_params=_cparams(),
        scratch_types=[pltpu.VMEM((BS * C,), jnp.float32),
                       pltpu.VMEM((L,), jnp.float32),
                       pltpu.VMEM((BLK, C), jnp.float32),
                       pltpu.VMEM((BLK, C), jnp.float32),
                       pltpu.VMEM((BLK,), jnp.int32),
                       pltpu.VMEM((BLK,), jnp.int32),
                       pltpu.SemaphoreType.DMA,
                       pltpu.SemaphoreType.DMA],
    )
    def k(feats_hbm, bidx_hbm, part_hbm, cnt_hbm,
          acc, cnt, fb0, fb1, bb0, bb1, sem0, sem1):
        zero = jnp.zeros((L,), jnp.float32)

        @pl.loop(0, BS * C, step=L)
        def _(i):
            acc[pl.ds(i, L)] = zero

        cnt[...] = zero
        iota = lax.iota(jnp.int32, L)
        one = jnp.float32(1.0)
        zf = jnp.float32(0.0)
        wid = _wid()
        nb = _nblocks_for(wid, NBLK_SC)
        head = TCA // BLK

        def start_in(j, fb, bb, sem):
            bid = wid + j * NW
            base = pl.multiple_of(
                jnp.where(bid < head, bid * BLK, TCB + (bid - head) * BLK), 8)
            pltpu.async_copy(feats_hbm.at[pl.ds(base, BLK)], fb, sem)
            pltpu.async_copy(bidx_hbm.at[pl.ds(base, BLK)], bb, sem)

        def wait_in(fb, bb, sem):
            pltpu.make_async_copy(feats_hbm.at[pl.ds(0, BLK)], fb, sem).wait()
            pltpu.make_async_copy(bidx_hbm.at[pl.ds(0, BLK)], bb, sem).wait()

        def compute(fb, bb):
            mn, mx = _block_minmax(bb)

            def fast():
                off = mx * C
                for half in range(2):
                    hof = half * HCSL * L

                    def rbody(r, sums):
                        return tuple(
                            sums[t] + fb[r, pl.ds(hof + t * L, L)]
                            for t in range(HCSL))

                    sums = lax.fori_loop(
                        0, BLK, rbody, (zero,) * HCSL)
                    for t in range(HCSL):
                        plsc.addupdate(acc.at[pl.ds(off + hof + t * L, L)],
                                       sums[t])
                plsc.addupdate(cnt.at[pl.ds(0, L)],
                               jnp.where(iota == mx, jnp.float32(BLK), zf))

            def slow():
                @pl.loop(0, BLK)
                def _(r):
                    b16 = plsc.load_gather(_fr(bb), [_I0(r, jnp.int32)])
                    addr = b16 * C + iota
                    for j in range(CSL):
                        plsc.addupdate_scatter(_fr(acc), [addr + (j * L)],
                                               fb[r, pl.ds(j * L, L)])
                    plsc.addupdate(cnt.at[pl.ds(0, L)],
                                   jnp.where(iota == b16, one, zf))

            lax.cond(mn == mx, fast, slow)

        start_in(0, fb0, bb0, sem0)
        start_in(1, fb1, bb1, sem1)

        @pl.loop(0, nb // 2)
        def _(p):
            j0 = 2 * p
            wait_in(fb0, bb0, sem0)
            compute(fb0, bb0)

            @pl.when(j0 + 2 < nb)
            def _():
                start_in(j0 + 2, fb0, bb0, sem0)

            wait_in(fb1, bb1, sem1)
            compute(fb1, bb1)

            @pl.when(j0 + 3 < nb)
            def _():
                start_in(j0 + 3, fb1, bb1, sem1)

        @pl.when(nb % 2 == 1)
        def _():
            wait_in(fb0, bb0, sem0)
            compute(fb0, bb0)

        pbase = pl.multiple_of(wid * (BS * C), 8)
        pltpu.sync_copy(acc, part_hbm.at[pl.ds(pbase, BS * C)])
        cbase = pl.multiple_of(wid * L, 8)
        pltpu.sync_copy(cnt, cnt_hbm.at[pl.ds(cbase, L)])

    return k(feats, bidx1)


def _pool_tc(feats, bidx2):
    def body(b_ref, f_ref, part_ref, cnt_ref, acc, cacc):
        i = pl.program_id(0)

        @pl.when(i == 0)
        def _():
            acc[...] = jnp.zeros_like(acc)
            cacc[...] = jnp.zeros_like(cacc)

        fx = f_ref[...]
        for b in range(BS):
            mask = (b_ref[...] == b).astype(jnp.float32)
            row = lax.dot_general(mask, fx, (((1,), (0,)), ((), ())),
                                  preferred_element_type=jnp.float32)
            acc[pl.ds(b, 1), :] += row
            cacc[pl.ds(b, 1), :] += jnp.sum(mask)

        @pl.when(i == TGRID - 1)
        def _():
            part_ref[...] = acc[...]
            cnt_ref[...] = cacc[...]

    return pl.pallas_call(
        body,
        grid=(TGRID,),
        in_specs=[pl.BlockSpec((1, TBLK), lambda i: (0, TCA // TBLK + i)),
                  pl.BlockSpec((TBLK, C), lambda i: (TCA // TBLK + i, 0))],
        out_specs=[pl.BlockSpec((BS, C), lambda i: (0, 0)),
                   pl.BlockSpec((BS, HID), lambda i: (0, 0))],
        out_shape=[jax.ShapeDtypeStruct((BS, C), jnp.float32),
                   jax.ShapeDtypeStruct((BS, HID), jnp.float32)],
        scratch_shapes=[pltpu.VMEM((BS, C), jnp.float32),
                        pltpu.VMEM((BS, HID), jnp.float32)],
    )(bidx2, feats)


def _mlp_tc(part, cnt, tc_part, tc_cnt, fc1_w, fc1_b, fc2_w, fc2_b):
    def body(part_ref, cnt_ref, tcp_ref, tcc_ref,
             w1_ref, b1_ref, w2_ref, b2_ref, se_ref):
        pooled = jnp.sum(
            part_ref[...].reshape(NW, BS, C), axis=0) + tcp_ref[...]
        counts = (jnp.sum(cnt_ref[...], axis=0)[:BS]
                  + jnp.sum(tcc_ref[...], axis=1) * (1.0 / HID))
        avg = pooled * (1.0 / jnp.maximum(counts, 1.0))[:, None]
        h = lax.dot_general(avg, w1_ref[...],
                            (((1,), (1,)), ((), ())),
                            preferred_element_type=jnp.float32)
        h = jnp.maximum(h + b1_ref[...][None, :], 0.0)
        s = lax.dot_general(h, w2_ref[...],
                            (((1,), (1,)), ((), ())),
                            preferred_element_type=jnp.float32)
        s = s + b2_ref[...][None, :]
        se_ref[...] = 1.0 / (1.0 + jnp.exp(-s))

    return pl.pallas_call(
        body,
        out_shape=jax.ShapeDtypeStruct((BS, C), jnp.float32),
    )(part.reshape(NW, BS * C), cnt.reshape(NW, L), tc_part, tc_cnt,
      fc1_w, fc1_b, fc2_w, fc2_b)


def _scale(feats, bidx1, se_flat):
    @functools.partial(
        pl.kernel,
        out_type=jax.ShapeDtypeStruct((N, C), jnp.float32),
        mesh=_mesh(),
        compiler_params=_cparams(),
        scratch_types=[pltpu.VMEM((C * BS,), jnp.float32),
                       pltpu.VMEM((BLK, C), jnp.float32),
                       pltpu.VMEM((BLK, C), jnp.float32),
                       pltpu.VMEM((BLK, C), jnp.float32),
                       pltpu.VMEM((BLK,), jnp.int32),
                       pltpu.VMEM((BLK,), jnp.int32),
                       pltpu.VMEM((BLK,), jnp.int32),
                       pltpu.SemaphoreType.DMA,
                       pltpu.SemaphoreType.DMA,
                       pltpu.SemaphoreType.DMA,
                       pltpu.SemaphoreType.DMA,
                       pltpu.SemaphoreType.DMA,
                       pltpu.SemaphoreType.DMA],
    )
    def k(feats_hbm, bidx_hbm, se_hbm, out_hbm,
          sev, fb0, fb1, fb2, bb0, bb1, bb2, si0, si1, si2, so0, so1, so2):
        iota = lax.iota(jnp.int32, L)
        wid = _wid()
        nb = _nblocks_for(wid)

        pltpu.sync_copy(se_hbm, sev)

        bufs = ((fb0, bb0, si0, so0),
                (fb1, bb1, si1, so1),
                (fb2, bb2, si2, so2))

        def start_in(j, buf):
            fb, bb, si, _so = buf
            base = pl.multiple_of((wid + j * NW) * BLK, 8)
            pltpu.async_copy(feats_hbm.at[pl.ds(base, BLK)], fb, si)
            pltpu.async_copy(bidx_hbm.at[pl.ds(base, BLK)], bb, si)

        def wait_in(buf):
            fb, bb, si, _so = buf
            pltpu.make_async_copy(feats_hbm.at[pl.ds(0, BLK)], fb, si).wait()
            pltpu.make_async_copy(bidx_hbm.at[pl.ds(0, BLK)], bb, si).wait()

        def start_out(j, buf):
            fb, _bb, _si, so = buf
            base = pl.multiple_of((wid + j * NW) * BLK, 8)
            pltpu.async_copy(fb, out_hbm.at[pl.ds(base, BLK)], so)

        def wait_out(buf):
            fb, _bb, _si, so = buf
            pltpu.make_async_copy(feats_hbm.at[pl.ds(0, BLK)], fb, so).wait()

        def compute(buf):
            fb, bb = buf[0], buf[1]
            mn, mx = _block_minmax(bb)

            def fast():
                off = mx * C
                for half in range(2):
                    hof = half * HCSL * L
                    ses_regs = tuple(sev[pl.ds(off + hof + t * L, L)]
                                     for t in range(HCSL))

                    @pl.loop(0, BLK)
                    def _(r):
                        for t in range(HCSL):
                            sl = pl.ds(hof + t * L, L)
                            fb[r, sl] = fb[r, sl] * ses_regs[t]

            def slow():
                @pl.loop(0, BLK)
                def _(r):
                    b16 = plsc.load_gather(_fr(bb), [_I0(r, jnp.int32)])
                    addr = b16 * C + iota
                    for j in range(CSL):
                        sl = pl.ds(j * L, L)
                        s16 = plsc.load_gather(_fr(sev), [addr + (j * L)])
                        fb[r, sl] = fb[r, sl] * s16

            lax.cond(mn == mx, fast, slow)

        def step(j, k_static, t):
            cur = bufs[k_static]
            nxt = bufs[(k_static + 2) % 3]
            wait_in(cur)
            compute(cur)
            start_out(j, cur)
            if k_static == 0:
                @pl.when(t > 0)
                def _():
                    wait_out(nxt)
            else:
                wait_out(nxt)

            @pl.when(j + 2 < nb)
            def _():
                start_in(j + 2, nxt)

        start_in(0, bufs[0])
        start_in(1, bufs[1])

        @pl.loop(0, nb // 3)
        def _(t):
            j0 = 3 * t
            step(j0, 0, t)
            step(j0 + 1, 1, t)
            step(j0 + 2, 2, t)

        @pl.when(nb % 3 == 1)
        def _():
            j = nb - 1
            cur = bufs[0]
            wait_in(cur)
            compute(cur)
            start_out(j, cur)
            wait_out(bufs[2])
            wait_out(bufs[0])

        @pl.when(nb % 3 == 0)
        def _():
            wait_out(bufs[2])

    return k(feats, bidx1, se_flat)


def kernel(feats, fc1_w, fc1_b, fc2_w, fc2_b, bidx, batch_size):
    del batch_size
    bidx1 = bidx.astype(jnp.int32).reshape(N)
    part, cnt = _pool(feats, bidx1)
    tc_part, tc_cnt = _pool_tc(feats, bidx1.reshape(1, N))
    se = _mlp_tc(part, cnt, tc_part, tc_cnt, fc1_w, fc1_b, fc2_w, fc2_b)
    return _scale(feats, bidx1, se.reshape(BS * C))
